# baseline (speedup 1.0000x reference)
import functools

import jax
import jax.numpy as jnp
from jax import lax
from jax.experimental import pallas as pl
from jax.experimental.pallas import tpu as pltpu
from jax.experimental.pallas import tpu_sc as plsc

_CHUNK = 64
_NBUF = 5
_OFF = 2


@functools.lru_cache(maxsize=None)
def _sc_concat_gather(n_rows: int, hidden: int):
    info = plsc.get_sparse_core_info()
    nc, ns = info.num_cores, info.num_subcores
    nw = nc * ns
    assert n_rows % (nw * _CHUNK) == 0
    rows_w = n_rows // nw
    n_chunks = rows_w // _CHUNK
    assert n_chunks % _NBUF == 0
    n_groups = n_chunks // _NBUF

    mesh = plsc.VectorSubcoreMesh(core_axis_name="c", subcore_axis_name="s")

    @functools.partial(
        pl.kernel,
        mesh=mesh,
        out_type=jax.ShapeDtypeStruct((n_rows, 2 * hidden), jnp.float32),
        scratch_types=[
            pltpu.VMEM((n_chunks, _CHUNK), jnp.int32),
            pltpu.VMEM((_NBUF, _CHUNK, 2 * hidden), jnp.float32),
        ]
        + [pltpu.SemaphoreType.DMA] * (3 * _NBUF),
    )
    def k(emb_hbm, ids_hbm, table_hbm, out_hbm, idx_v, bufs, *sems):
        sem_g = sems[:_NBUF]
        sem_e = sems[_NBUF : 2 * _NBUF]
        sem_s = sems[2 * _NBUF :]
        wid = lax.axis_index("s") * nc + lax.axis_index("c")
        base = wid * rows_w

        pltpu.sync_copy(ids_hbm.at[wid], idx_v)

        def gather(j, b):
            return pltpu.make_async_copy(
                table_hbm.at[idx_v.at[j]],
                bufs.at[b, :, pl.ds(hidden, hidden)],
                sem_g[b],
            )

        def emb_in(j, b):
            return pltpu.make_async_copy(
                emb_hbm.at[pl.ds(base + j * _CHUNK, _CHUNK), :],
                bufs.at[b, :, pl.ds(0, hidden)],
                sem_e[b],
            )

        def store(j, b):
            return pltpu.make_async_copy(
                bufs.at[b],
                out_hbm.at[pl.ds(base + j * _CHUNK, _CHUNK), :],
                sem_s[b],
            )

        def group(g, carry):
            for b in range(_NBUF):
                t = g * _NBUF + b
                @pl.when(g >= 1)
                def _():
                    store(t - _NBUF, b).wait()

                gather(t, b).start()
                emb_in(t, b).start()
                u = t - _OFF
                bb = (b - _OFF) % _NBUF

                @pl.when(u >= 0)
                def _():
                    gather(u, bb).wait()
                    emb_in(u, bb).wait()
                    store(u, bb).start()

            return carry

        lax.fori_loop(0, n_groups, group, 0)

        for u in range(n_chunks - _OFF, n_chunks):
            bb = u % _NBUF
            gather(u, bb).wait()
            emb_in(u, bb).wait()
            store(u, bb).start()
        for u in range(n_chunks - _NBUF, n_chunks):
            bb = u % _NBUF
            store(u, bb).wait()

    return k


def kernel(batch_mention_emb, mention_type_ids, embedding_table):
    b, l, h = batch_mention_emb.shape
    n = b * l
    nw = 32
    emb2d = batch_mention_emb.reshape(n, h)
    ids3d = (
        mention_type_ids.reshape(-1)
        .astype(jnp.int32)
        .reshape(nw, n // (nw * _CHUNK), _CHUNK)
    )
    out2d = _sc_concat_gather(n, h)(emb2d, ids3d, embedding_table)
    return out2d.reshape(b, l, 2 * h)


# speedup vs baseline: 1.0018x; 1.0018x over previous
import functools

import jax
import jax.numpy as jnp
from jax import lax
from jax.experimental import pallas as pl
from jax.experimental.pallas import tpu as pltpu
from jax.experimental.pallas import tpu_sc as plsc

_CHUNK = 128
_NBUF = 2
_OFF = 1


@functools.lru_cache(maxsize=None)
def _sc_concat_gather(n_rows: int, hidden: int):
    info = plsc.get_sparse_core_info()
    nc, ns = info.num_cores, info.num_subcores
    nw = nc * ns
    assert n_rows % (nw * _CHUNK) == 0
    rows_w = n_rows // nw
    n_chunks = rows_w // _CHUNK
    assert n_chunks % _NBUF == 0
    n_groups = n_chunks // _NBUF

    mesh = plsc.VectorSubcoreMesh(core_axis_name="c", subcore_axis_name="s")

    @functools.partial(
        pl.kernel,
        mesh=mesh,
        out_type=jax.ShapeDtypeStruct((n_rows, 2 * hidden), jnp.float32),
        scratch_types=[
            pltpu.VMEM((n_chunks, _CHUNK), jnp.int32),
            pltpu.VMEM((_NBUF, _CHUNK, 2 * hidden), jnp.float32),
        ]
        + [pltpu.SemaphoreType.DMA] * (3 * _NBUF),
    )
    def k(emb_hbm, ids_hbm, table_hbm, out_hbm, idx_v, bufs, *sems):
        sem_g = sems[:_NBUF]
        sem_e = sems[_NBUF : 2 * _NBUF]
        sem_s = sems[2 * _NBUF :]
        wid = lax.axis_index("s") * nc + lax.axis_index("c")
        base = wid * rows_w

        pltpu.sync_copy(ids_hbm.at[wid], idx_v)

        def gather(j, b):
            return pltpu.make_async_copy(
                table_hbm.at[idx_v.at[j]],
                bufs.at[b, :, pl.ds(hidden, hidden)],
                sem_g[b],
            )

        def emb_in(j, b):
            return pltpu.make_async_copy(
                emb_hbm.at[pl.ds(base + j * _CHUNK, _CHUNK), :],
                bufs.at[b, :, pl.ds(0, hidden)],
                sem_e[b],
            )

        def store(j, b):
            return pltpu.make_async_copy(
                bufs.at[b],
                out_hbm.at[pl.ds(base + j * _CHUNK, _CHUNK), :],
                sem_s[b],
            )

        def group(g, carry):
            for b in range(_NBUF):
                t = g * _NBUF + b
                @pl.when(g >= 1)
                def _():
                    store(t - _NBUF, b).wait()

                gather(t, b).start()
                emb_in(t, b).start()
                u = t - _OFF
                bb = (b - _OFF) % _NBUF

                @pl.when(u >= 0)
                def _():
                    gather(u, bb).wait()
                    emb_in(u, bb).wait()
                    store(u, bb).start()

            return carry

        lax.fori_loop(0, n_groups, group, 0)

        for u in range(n_chunks - _OFF, n_chunks):
            bb = u % _NBUF
            gather(u, bb).wait()
            emb_in(u, bb).wait()
            store(u, bb).start()
        for u in range(n_chunks - _NBUF, n_chunks):
            bb = u % _NBUF
            store(u, bb).wait()

    return k


def kernel(batch_mention_emb, mention_type_ids, embedding_table):
    b, l, h = batch_mention_emb.shape
    n = b * l
    nw = 32
    emb2d = batch_mention_emb.reshape(n, h)
    ids3d = (
        mention_type_ids.reshape(-1)
        .astype(jnp.int32)
        .reshape(nw, n // (nw * _CHUNK), _CHUNK)
    )
    out2d = _sc_concat_gather(n, h)(emb2d, ids3d, embedding_table)
    return out2d.reshape(b, l, 2 * h)


# speedup vs baseline: 1.7993x; 1.7961x over previous
import functools

import jax
import jax.numpy as jnp
from jax import lax
from jax.experimental import pallas as pl
from jax.experimental.pallas import tpu as pltpu
from jax.experimental.pallas import tpu_sc as plsc

_NBUF = 8
_OFF = 2


@functools.lru_cache(maxsize=None)
def _sc_concat_gather(n_batch: int, seq: int, hidden: int):
    info = plsc.get_sparse_core_info()
    nc, ns = info.num_cores, info.num_subcores
    nw = nc * ns
    assert n_batch % (nw * _NBUF) == 0
    bat_w = n_batch // nw
    n_groups = bat_w // _NBUF

    mesh = plsc.VectorSubcoreMesh(core_axis_name="c", subcore_axis_name="s")

    @functools.partial(
        pl.kernel,
        mesh=mesh,
        out_type=jax.ShapeDtypeStruct((n_batch, seq, 2 * hidden), jnp.float32),
        scratch_types=[
            pltpu.VMEM((bat_w, seq), jnp.int32),
            pltpu.VMEM((_NBUF, seq, 2 * hidden), jnp.float32),
        ]
        + [pltpu.SemaphoreType.DMA] * (3 * _NBUF),
    )
    def k(emb_hbm, ids_hbm, table_hbm, out_hbm, idx_v, bufs, *sems):
        sem_g = sems[:_NBUF]
        sem_e = sems[_NBUF : 2 * _NBUF]
        sem_s = sems[2 * _NBUF :]
        wid = lax.axis_index("s") * nc + lax.axis_index("c")
        base = wid * bat_w

        pltpu.sync_copy(ids_hbm.at[pl.ds(base, bat_w)], idx_v)

        def gather(j, b):
            return pltpu.make_async_copy(
                table_hbm.at[idx_v.at[j]],
                bufs.at[b, :, pl.ds(hidden, hidden)],
                sem_g[b],
            )

        def emb_in(j, b):
            return pltpu.make_async_copy(
                emb_hbm.at[base + j],
                bufs.at[b, :, pl.ds(0, hidden)],
                sem_e[b],
            )

        def store(j, b):
            return pltpu.make_async_copy(
                bufs.at[b],
                out_hbm.at[base + j],
                sem_s[b],
            )

        def group(g, carry):
            for b in range(_NBUF):
                t = g * _NBUF + b
                @pl.when(g >= 1)
                def _():
                    store(t - _NBUF, b).wait()

                gather(t, b).start()
                emb_in(t, b).start()
                u = t - _OFF
                bb = (b - _OFF) % _NBUF

                @pl.when(u >= 0)
                def _():
                    gather(u, bb).wait()
                    emb_in(u, bb).wait()
                    store(u, bb).start()

            return carry

        lax.fori_loop(0, n_groups, group, 0)

        for u in range(bat_w - _OFF, bat_w):
            bb = u % _NBUF
            gather(u, bb).wait()
            emb_in(u, bb).wait()
            store(u, bb).start()
        for u in range(bat_w - _NBUF, bat_w):
            bb = u % _NBUF
            store(u, bb).wait()

    return k


def kernel(batch_mention_emb, mention_type_ids, embedding_table):
    b, l, h = batch_mention_emb.shape
    ids = mention_type_ids.astype(jnp.int32)
    return _sc_concat_gather(b, l, h)(batch_mention_emb, ids, embedding_table)


# speedup vs baseline: 1.8029x; 1.0020x over previous
import functools

import jax
import jax.numpy as jnp
from jax import lax
from jax.experimental import pallas as pl
from jax.experimental.pallas import tpu as pltpu
from jax.experimental.pallas import tpu_sc as plsc

_NB = 4
_NBUF = 2
_OFF = 1


@functools.lru_cache(maxsize=None)
def _sc_concat_gather(n_batch: int, seq: int, hidden: int):
    info = plsc.get_sparse_core_info()
    nc, ns = info.num_cores, info.num_subcores
    nw = nc * ns
    assert n_batch % (nw * _NB * _NBUF) == 0
    bat_w = n_batch // nw
    n_chunks = bat_w // _NB
    n_groups = n_chunks // _NBUF

    mesh = plsc.VectorSubcoreMesh(core_axis_name="c", subcore_axis_name="s")

    @functools.partial(
        pl.kernel,
        mesh=mesh,
        out_type=jax.ShapeDtypeStruct((n_batch, seq, 2 * hidden), jnp.float32),
        scratch_types=[
            pltpu.VMEM((bat_w, seq), jnp.int32),
            pltpu.VMEM((_NBUF, _NB, seq, 2 * hidden), jnp.float32),
        ]
        + [pltpu.SemaphoreType.DMA] * (3 * _NBUF),
    )
    def k(emb_hbm, ids_hbm, table_hbm, out_hbm, idx_v, bufs, *sems):
        sem_g = sems[:_NBUF]
        sem_e = sems[_NBUF : 2 * _NBUF]
        sem_s = sems[2 * _NBUF :]
        wid = lax.axis_index("s") * nc + lax.axis_index("c")
        base = wid * bat_w

        pltpu.sync_copy(ids_hbm.at[pl.ds(base, bat_w)], idx_v)

        def gathers_start(j, b):
            for q in range(_NB):
                pltpu.make_async_copy(
                    table_hbm.at[idx_v.at[j * _NB + q]],
                    bufs.at[b, q, :, pl.ds(hidden, hidden)],
                    sem_g[b],
                ).start()

        def gathers_wait(j, b):
            for q in range(_NB):
                pltpu.make_async_copy(
                    table_hbm.at[idx_v.at[j * _NB + q]],
                    bufs.at[b, q, :, pl.ds(hidden, hidden)],
                    sem_g[b],
                ).wait()

        def emb_in(j, b):
            return pltpu.make_async_copy(
                emb_hbm.at[pl.ds(base + j * _NB, _NB)],
                bufs.at[b, :, :, pl.ds(0, hidden)],
                sem_e[b],
            )

        def store(j, b):
            return pltpu.make_async_copy(
                bufs.at[b],
                out_hbm.at[pl.ds(base + j * _NB, _NB)],
                sem_s[b],
            )

        def group(g, carry):
            for b in range(_NBUF):
                t = g * _NBUF + b
                @pl.when(g >= 1)
                def _():
                    store(t - _NBUF, b).wait()

                gathers_start(t, b)
                emb_in(t, b).start()
                u = t - _OFF
                bb = (b - _OFF) % _NBUF

                @pl.when(u >= 0)
                def _():
                    gathers_wait(u, bb)
                    emb_in(u, bb).wait()
                    store(u, bb).start()

            return carry

        lax.fori_loop(0, n_groups, group, 0)

        for u in range(n_chunks - _OFF, n_chunks):
            bb = u % _NBUF
            gathers_wait(u, bb)
            emb_in(u, bb).wait()
            store(u, bb).start()
        for u in range(n_chunks - _NBUF, n_chunks):
            bb = u % _NBUF
            store(u, bb).wait()

    return k


def kernel(batch_mention_emb, mention_type_ids, embedding_table):
    b, l, h = batch_mention_emb.shape
    ids = mention_type_ids.astype(jnp.int32)
    return _sc_concat_gather(b, l, h)(batch_mention_emb, ids, embedding_table)


# speedup vs baseline: 1.8039x; 1.0006x over previous
import functools

import jax
import jax.numpy as jnp
from jax import lax
from jax.experimental import pallas as pl
from jax.experimental.pallas import tpu as pltpu
from jax.experimental.pallas import tpu_sc as plsc

_NBUF = 8
_OFF = 4


@functools.lru_cache(maxsize=None)
def _sc_concat_gather(n_batch: int, seq: int, hidden: int):
    info = plsc.get_sparse_core_info()
    nc, ns = info.num_cores, info.num_subcores
    nw = nc * ns
    assert n_batch % (nw * _NBUF) == 0
    bat_w = n_batch // nw
    n_groups = bat_w // _NBUF

    mesh = plsc.VectorSubcoreMesh(core_axis_name="c", subcore_axis_name="s")

    @functools.partial(
        pl.kernel,
        mesh=mesh,
        out_type=jax.ShapeDtypeStruct((n_batch, seq, 2 * hidden), jnp.float32),
        scratch_types=[
            pltpu.VMEM((bat_w, seq), jnp.int32),
            pltpu.VMEM((_NBUF, seq, 2 * hidden), jnp.float32),
        ]
        + [pltpu.SemaphoreType.DMA] * (3 * _NBUF),
    )
    def k(emb_hbm, ids_hbm, table_hbm, out_hbm, idx_v, bufs, *sems):
        sem_g = sems[:_NBUF]
        sem_e = sems[_NBUF : 2 * _NBUF]
        sem_s = sems[2 * _NBUF :]
        wid = lax.axis_index("s") * nc + lax.axis_index("c")
        base = wid * bat_w

        pltpu.sync_copy(ids_hbm.at[pl.ds(base, bat_w)], idx_v)

        def gather(j, b):
            return pltpu.make_async_copy(
                table_hbm.at[idx_v.at[j]],
                bufs.at[b, :, pl.ds(hidden, hidden)],
                sem_g[b],
            )

        def emb_in(j, b):
            return pltpu.make_async_copy(
                emb_hbm.at[base + j],
                bufs.at[b, :, pl.ds(0, hidden)],
                sem_e[b],
            )

        def store(j, b):
            return pltpu.make_async_copy(
                bufs.at[b],
                out_hbm.at[base + j],
                sem_s[b],
            )

        def group(g, carry):
            for b in range(_NBUF):
                t = g * _NBUF + b
                @pl.when(g >= 1)
                def _():
                    store(t - _NBUF, b).wait()

                gather(t, b).start()
                emb_in(t, b).start()
                u = t - _OFF
                bb = (b - _OFF) % _NBUF

                @pl.when(u >= 0)
                def _():
                    gather(u, bb).wait()
                    emb_in(u, bb).wait()
                    store(u, bb).start()

            return carry

        lax.fori_loop(0, n_groups, group, 0)

        for u in range(bat_w - _OFF, bat_w):
            bb = u % _NBUF
            gather(u, bb).wait()
            emb_in(u, bb).wait()
            store(u, bb).start()
        for u in range(bat_w - _NBUF, bat_w):
            bb = u % _NBUF
            store(u, bb).wait()

    return k


def kernel(batch_mention_emb, mention_type_ids, embedding_table):
    b, l, h = batch_mention_emb.shape
    ids = mention_type_ids.astype(jnp.int32)
    return _sc_concat_gather(b, l, h)(batch_mention_emb, ids, embedding_table)
